# baseline (speedup 1.0000x reference)
import functools

import jax
import jax.numpy as jnp
from jax import lax
from jax.experimental import pallas as pl
from jax.experimental.pallas import tpu as pltpu
from jax.experimental.pallas import tpu_sc as plsc

_DIM = 1024
_NE = 8
_NG = 4
_BLK = 2048

_NC = 1
_NS = 16
_NW = _NC * _NS
_L = 16


def _tc_scores_body(x_ref, w_ref, st_ref):
    x = x_ref[...]
    w = w_ref[...]
    raw = jax.lax.dot_general(
        x, w, (((1,), (1,)), ((), ())),
        preferred_element_type=jnp.float32)
    st_ref[...] = jax.nn.sigmoid(raw).T


def _tc_scores(x, w):
    n_tok = x.shape[0]
    return pl.pallas_call(
        _tc_scores_body,
        grid=(n_tok // _BLK,),
        in_specs=[
            pl.BlockSpec((_BLK, _DIM), lambda i: (i, 0)),
            pl.BlockSpec((_NE, _DIM), lambda i: (0, 0)),
        ],
        out_specs=pl.BlockSpec((_NE, _BLK), lambda i: (0, i)),
        out_shape=jax.ShapeDtypeStruct((_NE, n_tok), jnp.float32),
    )(x, w)


def _sc_route_body(tw, st_hbm, w0_hbm, w1_hbm, i0_hbm, i1_hbm,
                   s_v, w0_v, w1_v, i0_v, i1_v):
    wid = lax.axis_index("s") * _NC + lax.axis_index("c")
    base = wid * tw
    pltpu.sync_copy(st_hbm.at[:, pl.ds(base, tw)], s_v)

    neg = jnp.full((_L,), -jnp.inf, jnp.float32)
    one = jnp.full((_L,), 1, jnp.int32)
    zero = jnp.full((_L,), 0, jnp.int32)
    def step(t, carry):
        off = t * _L
        s = [s_v[e, pl.ds(off, _L)] for e in range(_NE)]
        g = [jnp.maximum(s[2 * j], s[2 * j + 1]) for j in range(_NG)]
        cnt = [zero] * _NG
        for j in range(_NG):
            for k in range(j + 1, _NG):
                jk = g[j] >= g[k]
                cnt[k] = cnt[k] + jnp.where(jk, one, zero)
                cnt[j] = cnt[j] + jnp.where(jk, zero, one)
        sel = [cnt[j] < 2 for j in range(_NG)]
        m = [jnp.where(sel[e // 2], s[e], neg) for e in range(_NE)]
        best, bidx = m[0], zero
        for e in range(1, _NE):
            c = m[e] > best
            best = jnp.where(c, m[e], best)
            bidx = jnp.where(c, jnp.full((_L,), e, jnp.int32), bidx)
        m2 = [jnp.where(bidx == e, neg, m[e]) for e in range(_NE)]
        best2, bidx2 = m2[0], zero
        for e in range(1, _NE):
            c = m2[e] > best2
            best2 = jnp.where(c, m2[e], best2)
            bidx2 = jnp.where(c, jnp.full((_L,), e, jnp.int32), bidx2)
        tot = best + best2
        w0_v[pl.ds(off, _L)] = best / tot
        w1_v[pl.ds(off, _L)] = best2 / tot
        i0_v[pl.ds(off, _L)] = bidx
        i1_v[pl.ds(off, _L)] = bidx2
        return carry

    lax.fori_loop(0, tw // _L, step, 0)

    pltpu.sync_copy(w0_v, w0_hbm.at[pl.ds(base, tw)])
    pltpu.sync_copy(w1_v, w1_hbm.at[pl.ds(base, tw)])
    pltpu.sync_copy(i0_v, i0_hbm.at[pl.ds(base, tw)])
    pltpu.sync_copy(i1_v, i1_hbm.at[pl.ds(base, tw)])


def _sc_route(st):
    n_tok = st.shape[1]
    tw = n_tok // _NW
    mesh = plsc.VectorSubcoreMesh(core_axis_name="c", subcore_axis_name="s", num_cores=1)
    f = functools.partial(
        pl.kernel,
        out_type=[
            jax.ShapeDtypeStruct((n_tok,), jnp.float32),
            jax.ShapeDtypeStruct((n_tok,), jnp.float32),
            jax.ShapeDtypeStruct((n_tok,), jnp.int32),
            jax.ShapeDtypeStruct((n_tok,), jnp.int32),
        ],
        mesh=mesh,
        scratch_types=[
            pltpu.VMEM((_NE, tw), jnp.float32),
            pltpu.VMEM((tw,), jnp.float32),
            pltpu.VMEM((tw,), jnp.float32),
            pltpu.VMEM((tw,), jnp.int32),
            pltpu.VMEM((tw,), jnp.int32),
        ],
    )(functools.partial(_sc_route_body, tw))
    return f(st)


@jax.jit
def kernel(x, weight):
    st = _tc_scores(x, weight)
    w0, w1, i0, i1 = _sc_route(st)
    return jnp.stack([w0, w1], axis=1), jnp.stack([i0, i1], axis=1)


# speedup vs baseline: 1.0200x; 1.0200x over previous
import functools

import jax
import jax.numpy as jnp
from jax import lax
from jax.experimental import pallas as pl
from jax.experimental.pallas import tpu as pltpu
from jax.experimental.pallas import tpu_sc as plsc

_DIM = 1024
_NE = 8
_NG = 4
_BLK = 2048

_NC = 2
_NS = 16
_NW = _NC * _NS
_L = 16


def _tc_scores_body(x_ref, w_ref, st_ref):
    x = x_ref[...]
    w = w_ref[...]
    raw = jax.lax.dot_general(
        x, w, (((1,), (1,)), ((), ())),
        preferred_element_type=jnp.float32)
    st_ref[...] = jax.nn.sigmoid(raw).T


def _tc_scores(x, w):
    n_tok = x.shape[0]
    return pl.pallas_call(
        _tc_scores_body,
        grid=(n_tok // _BLK,),
        in_specs=[
            pl.BlockSpec((_BLK, _DIM), lambda i: (i, 0)),
            pl.BlockSpec((_NE, _DIM), lambda i: (0, 0)),
        ],
        out_specs=pl.BlockSpec((_NE, _BLK), lambda i: (0, i)),
        out_shape=jax.ShapeDtypeStruct((_NE, n_tok), jnp.float32),
    )(x, w)


def _sc_route_body(tw, st_hbm, w0_hbm, w1_hbm, i0_hbm, i1_hbm,
                   s_v, w0_v, w1_v, i0_v, i1_v):
    wid = lax.axis_index("s") * _NC + lax.axis_index("c")
    base = wid * tw
    pltpu.sync_copy(st_hbm.at[:, pl.ds(base, tw)], s_v)

    neg = jnp.full((_L,), -jnp.inf, jnp.float32)
    one = jnp.full((_L,), 1, jnp.int32)
    zero = jnp.full((_L,), 0, jnp.int32)
    def step(t, carry):
        off = t * _L
        s = [s_v[e, pl.ds(off, _L)] for e in range(_NE)]
        g = [jnp.maximum(s[2 * j], s[2 * j + 1]) for j in range(_NG)]
        cnt = [zero] * _NG
        for j in range(_NG):
            for k in range(j + 1, _NG):
                jk = g[j] >= g[k]
                cnt[k] = cnt[k] + jnp.where(jk, one, zero)
                cnt[j] = cnt[j] + jnp.where(jk, zero, one)
        sel = [cnt[j] < 2 for j in range(_NG)]
        m = [jnp.where(sel[e // 2], s[e], neg) for e in range(_NE)]
        best, bidx = m[0], zero
        for e in range(1, _NE):
            c = m[e] > best
            best = jnp.where(c, m[e], best)
            bidx = jnp.where(c, jnp.full((_L,), e, jnp.int32), bidx)
        m2 = [jnp.where(bidx == e, neg, m[e]) for e in range(_NE)]
        best2, bidx2 = m2[0], zero
        for e in range(1, _NE):
            c = m2[e] > best2
            best2 = jnp.where(c, m2[e], best2)
            bidx2 = jnp.where(c, jnp.full((_L,), e, jnp.int32), bidx2)
        tot = best + best2
        w0_v[pl.ds(off, _L)] = best / tot
        w1_v[pl.ds(off, _L)] = best2 / tot
        i0_v[pl.ds(off, _L)] = bidx
        i1_v[pl.ds(off, _L)] = bidx2
        return carry

    lax.fori_loop(0, tw // _L, step, 0)

    pltpu.sync_copy(w0_v, w0_hbm.at[pl.ds(base, tw)])
    pltpu.sync_copy(w1_v, w1_hbm.at[pl.ds(base, tw)])
    pltpu.sync_copy(i0_v, i0_hbm.at[pl.ds(base, tw)])
    pltpu.sync_copy(i1_v, i1_hbm.at[pl.ds(base, tw)])


def _sc_route(st):
    n_tok = st.shape[1]
    tw = n_tok // _NW
    mesh = plsc.VectorSubcoreMesh(core_axis_name="c", subcore_axis_name="s")
    f = functools.partial(
        pl.kernel,
        out_type=[
            jax.ShapeDtypeStruct((n_tok,), jnp.float32),
            jax.ShapeDtypeStruct((n_tok,), jnp.float32),
            jax.ShapeDtypeStruct((n_tok,), jnp.int32),
            jax.ShapeDtypeStruct((n_tok,), jnp.int32),
        ],
        mesh=mesh,
        scratch_types=[
            pltpu.VMEM((_NE, tw), jnp.float32),
            pltpu.VMEM((tw,), jnp.float32),
            pltpu.VMEM((tw,), jnp.float32),
            pltpu.VMEM((tw,), jnp.int32),
            pltpu.VMEM((tw,), jnp.int32),
        ],
    )(functools.partial(_sc_route_body, tw))
    return f(st)


@jax.jit
def kernel(x, weight):
    st = _tc_scores(x, weight)
    w0, w1, i0, i1 = _sc_route(st)
    return jnp.stack([w0, w1], axis=1), jnp.stack([i0, i1], axis=1)


# speedup vs baseline: 1.0231x; 1.0030x over previous
import functools

import jax
import jax.numpy as jnp
from jax import lax
from jax.experimental import pallas as pl
from jax.experimental.pallas import tpu as pltpu
from jax.experimental.pallas import tpu_sc as plsc

_DIM = 1024
_NE = 8
_NG = 4
_BLK = 2048

_NC = 2
_NS = 16
_NW = _NC * _NS
_L = 16


def _tc_scores_body(x_ref, w_ref, st_ref):
    x = x_ref[...]
    w = w_ref[...]
    raw = jax.lax.dot_general(
        x, w, (((1,), (1,)), ((), ())),
        preferred_element_type=jnp.float32)
    st_ref[...] = jax.nn.sigmoid(raw).T


def _tc_scores(x, w):
    n_tok = x.shape[0]
    return pl.pallas_call(
        _tc_scores_body,
        grid=(n_tok // _BLK,),
        in_specs=[
            pl.BlockSpec((_BLK, _DIM), lambda i: (i, 0)),
            pl.BlockSpec((_NE, _DIM), lambda i: (0, 0)),
        ],
        out_specs=pl.BlockSpec((_NE, _BLK), lambda i: (0, i)),
        out_shape=jax.ShapeDtypeStruct((_NE, n_tok), jnp.float32),
    )(x, w)


def _sc_route_body(tw, st_hbm, w0_hbm, w1_hbm, i0_hbm, i1_hbm,
                   s_v, w0_v, w1_v, i0_v, i1_v):
    wid = lax.axis_index("s") * _NC + lax.axis_index("c")
    base = wid * tw
    pltpu.sync_copy(st_hbm.at[:, pl.ds(base, tw)], s_v)

    neg = jnp.full((_L,), -jnp.inf, jnp.float32)
    one = jnp.full((_L,), 1, jnp.int32)
    zero = jnp.full((_L,), 0, jnp.int32)
    def step(t, carry):
        off = t * _L
        s = [s_v[e, pl.ds(off, _L)] for e in range(_NE)]
        g = [jnp.maximum(s[2 * j], s[2 * j + 1]) for j in range(_NG)]
        cnt = [zero] * _NG
        for j in range(_NG):
            for k in range(j + 1, _NG):
                jk = g[j] >= g[k]
                cnt[k] = cnt[k] + jnp.where(jk, one, zero)
                cnt[j] = cnt[j] + jnp.where(jk, zero, one)
        sel = [cnt[j] < 2 for j in range(_NG)]
        m = [jnp.where(sel[e // 2], s[e], neg) for e in range(_NE)]
        best, bidx = m[0], zero
        for e in range(1, _NE):
            c = m[e] > best
            best = jnp.where(c, m[e], best)
            bidx = jnp.where(c, jnp.full((_L,), e, jnp.int32), bidx)
        m2 = [jnp.where(bidx == e, neg, m[e]) for e in range(_NE)]
        best2, bidx2 = m2[0], zero
        for e in range(1, _NE):
            c = m2[e] > best2
            best2 = jnp.where(c, m2[e], best2)
            bidx2 = jnp.where(c, jnp.full((_L,), e, jnp.int32), bidx2)
        tot = best + best2
        w0_v[pl.ds(off, _L)] = best / tot
        w1_v[pl.ds(off, _L)] = best2 / tot
        i0_v[pl.ds(off, _L)] = bidx
        i1_v[pl.ds(off, _L)] = bidx2
        return carry

    plsc.parallel_loop(0, tw // _L, 1, unroll=4)(lambda t: step(t, None))

    pltpu.sync_copy(w0_v, w0_hbm.at[pl.ds(base, tw)])
    pltpu.sync_copy(w1_v, w1_hbm.at[pl.ds(base, tw)])
    pltpu.sync_copy(i0_v, i0_hbm.at[pl.ds(base, tw)])
    pltpu.sync_copy(i1_v, i1_hbm.at[pl.ds(base, tw)])


def _sc_route(st):
    n_tok = st.shape[1]
    tw = n_tok // _NW
    mesh = plsc.VectorSubcoreMesh(core_axis_name="c", subcore_axis_name="s")
    f = functools.partial(
        pl.kernel,
        out_type=[
            jax.ShapeDtypeStruct((n_tok,), jnp.float32),
            jax.ShapeDtypeStruct((n_tok,), jnp.float32),
            jax.ShapeDtypeStruct((n_tok,), jnp.int32),
            jax.ShapeDtypeStruct((n_tok,), jnp.int32),
        ],
        mesh=mesh,
        scratch_types=[
            pltpu.VMEM((_NE, tw), jnp.float32),
            pltpu.VMEM((tw,), jnp.float32),
            pltpu.VMEM((tw,), jnp.float32),
            pltpu.VMEM((tw,), jnp.int32),
            pltpu.VMEM((tw,), jnp.int32),
        ],
    )(functools.partial(_sc_route_body, tw))
    return f(st)


@jax.jit
def kernel(x, weight):
    st = _tc_scores(x, weight)
    w0, w1, i0, i1 = _sc_route(st)
    return jnp.stack([w0, w1], axis=1), jnp.stack([i0, i1], axis=1)
